# baseline (speedup 1.0000x reference)
import functools

import jax
import jax.numpy as jnp
from jax import lax
from jax.experimental import pallas as pl
from jax.experimental.pallas import tpu as pltpu
from jax.experimental.pallas import tpu_sc as plsc

TOPK = 2
TILE = 256
CBLK = 128
NC, NS = 2, 16


def _router_body(nb, ntiles, E, S, xf_ref, wr_ref,
                 g0_ref, g1_ref, p0_ref, p1_ref, eot_ref, act_ref,
                 zl_ref, al_ref, rank_ref):
    Tt = xf_ref.shape[0]
    xf = xf_ref[...]
    wr = wr_ref[...]
    logits = lax.dot_general(xf, wr, (((1,), (1,)), ((), ())),
                             preferred_element_type=jnp.float32)
    zl_ref[...] = E * jnp.mean(jnp.square(logits), keepdims=True)

    m = jnp.max(logits, axis=1, keepdims=True)
    ex = jnp.exp(logits - m)
    probs = ex / jnp.sum(ex, axis=1, keepdims=True)
    pmean = jnp.mean(probs, axis=0, keepdims=True)
    al_ref[...] = E * (E * jnp.mean(pmean * jnp.log(pmean + 1e-9),
                                    keepdims=True))

    iota_e = lax.broadcasted_iota(jnp.int32, (Tt, E), 1)
    m0 = jnp.max(probs, axis=1, keepdims=True)
    idx0 = jnp.min(jnp.where(probs == m0, iota_e, E), axis=1, keepdims=True)
    sel0 = iota_e == idx0
    probs1 = jnp.where(sel0, -1.0, probs)
    m1 = jnp.max(probs1, axis=1, keepdims=True)
    idx1 = jnp.min(jnp.where(probs1 == m1, iota_e, E), axis=1, keepdims=True)
    sel1 = iota_e == idx1
    ssum = m0 + m1
    ones16 = jnp.ones((1, 16), jnp.float32)
    g0_ref[...] = (m0 / ssum) * ones16
    g1_ref[...] = (m1 / ssum) * ones16

    c = sel0.astype(jnp.float32) + sel1.astype(jnp.float32)
    tri = (lax.broadcasted_iota(jnp.int32, (CBLK, CBLK), 0) >
           lax.broadcasted_iota(jnp.int32, (CBLK, CBLK), 1)).astype(jnp.float32)
    off = jnp.zeros((1, E), jnp.float32)
    for b in range(nb):
        cb = c[b * CBLK:(b + 1) * CBLK, :]
        rank_ref[b * CBLK:(b + 1) * CBLK, :] = lax.dot_general(
            tri, cb, (((1,), (0,)), ((), ())),
            preferred_element_type=jnp.float32) + off
        off = off + jnp.sum(cb, axis=0, keepdims=True)
    counts = off

    padded = jnp.ceil(counts / TILE) * TILE
    triE = (lax.broadcasted_iota(jnp.int32, (E, E), 0) <
            lax.broadcasted_iota(jnp.int32, (E, E), 1)).astype(jnp.float32)
    starts = lax.dot_general(padded, triE, (((1,), (0,)), ((), ())),
                             preferred_element_type=jnp.float32)
    ends = starts + padded

    rank = rank_ref[...]
    pos0 = jnp.sum((starts + rank) * sel0.astype(jnp.float32), axis=1,
                   keepdims=True)
    pos1 = jnp.sum((starts + rank) * sel1.astype(jnp.float32), axis=1,
                   keepdims=True)
    p0_ref[...] = pos0.astype(jnp.int32)
    p1_ref[...] = pos1.astype(jnp.int32)

    tile_start = (lax.broadcasted_iota(jnp.int32, (ntiles, 1), 0)
                  * TILE).astype(jnp.float32)
    eot = jnp.sum((tile_start >= ends).astype(jnp.int32), axis=1,
                  keepdims=True)
    eot_ref[...] = jnp.minimum(eot, E - 1)
    total = jnp.sum(padded, axis=1, keepdims=True)
    act_ref[...] = (tile_start < total).astype(jnp.int32)


def _router(xf, Wr, ntiles):
    Tt, _ = xf.shape
    E = Wr.shape[0]
    S = 1
    nb = Tt // CBLK
    f32, i32 = jnp.float32, jnp.int32
    outs = pl.pallas_call(
        functools.partial(_router_body, nb, ntiles, E, S),
        out_shape=[
            jax.ShapeDtypeStruct((Tt, 16), f32),
            jax.ShapeDtypeStruct((Tt, 16), f32),
            jax.ShapeDtypeStruct((Tt, 1), i32),
            jax.ShapeDtypeStruct((Tt, 1), i32),
            jax.ShapeDtypeStruct((ntiles, 1), i32),
            jax.ShapeDtypeStruct((ntiles, 1), i32),
            jax.ShapeDtypeStruct((1, 1), f32),
            jax.ShapeDtypeStruct((1, 1), f32),
        ],
        scratch_shapes=[pltpu.VMEM((Tt, E), f32)],
    )(xf, Wr)
    return outs


def _dispatch_body(per_w, xf_hbm, p0_hbm, p1_hbm, xs_hbm,
                   p0_v, p1_v, rows_v, sem, sem2, sem3):
    wid = lax.axis_index("s") * NC + lax.axis_index("c")
    base = wid * per_w
    l0 = pltpu.async_copy(p0_hbm.at[pl.ds(base, per_w)], p0_v, sem)
    l1 = pltpu.async_copy(p1_hbm.at[pl.ds(base, per_w)], p1_v, sem2)
    l2 = pltpu.async_copy(xf_hbm.at[pl.ds(base, per_w)], rows_v, sem3)
    l0.wait()
    l1.wait()
    l2.wait()
    c0 = pltpu.async_copy(rows_v, xs_hbm.at[p0_v], sem)
    c1 = pltpu.async_copy(rows_v, xs_hbm.at[p1_v], sem2)
    c0.wait()
    c1.wait()


def _dispatch(xf, pos0, pos1, M):
    Tt, C = xf.shape
    per_w = Tt // (NC * NS)
    mesh = plsc.VectorSubcoreMesh(core_axis_name="c", subcore_axis_name="s",
                                  num_cores=NC, num_subcores=NS)
    f = functools.partial(
        pl.kernel,
        out_type=jax.ShapeDtypeStruct((M, C), jnp.float32),
        mesh=mesh,
        scratch_types=[
            pltpu.VMEM((per_w,), jnp.int32),
            pltpu.VMEM((per_w,), jnp.int32),
            pltpu.VMEM((per_w, C), jnp.float32),
            pltpu.SemaphoreType.DMA,
            pltpu.SemaphoreType.DMA,
            pltpu.SemaphoreType.DMA,
        ],
    )(functools.partial(_dispatch_body, per_w))
    return f(xf, pos0, pos1)


def _mlp_body(S, eot_ref, act_ref, xs_ref, wgs_ref, wus_ref, wds_ref,
              wgr_ref, wur_ref, wdr_ref, ys_ref):
    i = pl.program_id(0)
    active = act_ref[i] > 0
    is_s = eot_ref[i] < S
    xs = xs_ref[...].astype(jnp.bfloat16)

    def run(wg_ref, wu_ref, wd_ref):
        wg = wg_ref[0].astype(jnp.bfloat16)
        wu = wu_ref[0].astype(jnp.bfloat16)
        wd = wd_ref[0].astype(jnp.bfloat16)
        a = lax.dot_general(xs, wg, (((1,), (1,)), ((), ())),
                            preferred_element_type=jnp.float32)
        b = lax.dot_general(xs, wu, (((1,), (1,)), ((), ())),
                            preferred_element_type=jnp.float32)
        h = ((a * jax.nn.sigmoid(a)) * b).astype(jnp.bfloat16)
        ys_ref[...] = lax.dot_general(h, wd, (((1,), (1,)), ((), ())),
                                      preferred_element_type=jnp.float32)

    @pl.when(jnp.logical_and(active, is_s))
    def _():
        run(wgs_ref, wus_ref, wds_ref)

    @pl.when(jnp.logical_and(active, jnp.logical_not(is_s)))
    def _():
        run(wgr_ref, wur_ref, wdr_ref)


def _mlp(xs, eot, act, Wg_s, Wu_s, Wd_s, Wg_r, Wu_r, Wd_r):
    M, C = xs.shape
    S, DFF, _ = Wg_s.shape
    R = Wg_r.shape[0]
    ntiles = M // TILE
    grid_spec = pltpu.PrefetchScalarGridSpec(
        num_scalar_prefetch=2,
        grid=(ntiles,),
        in_specs=[
            pl.BlockSpec((TILE, C), lambda i, eot, act: (i, 0)),
            pl.BlockSpec((1, DFF, C),
                         lambda i, eot, act: (jnp.minimum(eot[i], S - 1), 0, 0)),
            pl.BlockSpec((1, DFF, C),
                         lambda i, eot, act: (jnp.minimum(eot[i], S - 1), 0, 0)),
            pl.BlockSpec((1, C, DFF),
                         lambda i, eot, act: (jnp.minimum(eot[i], S - 1), 0, 0)),
            pl.BlockSpec((1, DFF, C),
                         lambda i, eot, act: (jnp.clip(eot[i] - S, 0, R - 1),
                                              0, 0)),
            pl.BlockSpec((1, DFF, C),
                         lambda i, eot, act: (jnp.clip(eot[i] - S, 0, R - 1),
                                              0, 0)),
            pl.BlockSpec((1, C, DFF),
                         lambda i, eot, act: (jnp.clip(eot[i] - S, 0, R - 1),
                                              0, 0)),
        ],
        out_specs=pl.BlockSpec((TILE, C), lambda i, eot, act: (i, 0)),
    )
    return pl.pallas_call(
        functools.partial(_mlp_body, S),
        grid_spec=grid_spec,
        out_shape=jax.ShapeDtypeStruct((M, C), jnp.float32),
        compiler_params=pltpu.CompilerParams(vmem_limit_bytes=63 << 20),
    )(eot, act, xs, Wg_s, Wu_s, Wd_s, Wg_r, Wu_r, Wd_r)


CH = 16


def _combine_body(per_w, C, ys_hbm, p0_hbm, p1_hbm, g0_hbm, g1_hbm, out_hbm,
                  p0_v, p1_v, g0_v, g1_v, r0_v, r1_v, o_v, sems0, sems1, semo):
    wid = lax.axis_index("s") * NC + lax.axis_index("c")
    nchunk = per_w // CH
    nvec = C // 16

    def start(k, s):
        base = wid * per_w + k * CH
        pltpu.sync_copy(p0_hbm.at[pl.ds(base, CH)], p0_v.at[s])
        pltpu.sync_copy(p1_hbm.at[pl.ds(base, CH)], p1_v.at[s])
        pltpu.sync_copy(g0_hbm.at[pl.ds(base, CH)], g0_v.at[s])
        pltpu.sync_copy(g1_hbm.at[pl.ds(base, CH)], g1_v.at[s])
        c0 = pltpu.async_copy(ys_hbm.at[p0_v.at[s]], r0_v.at[s], sems0.at[s])
        c1 = pltpu.async_copy(ys_hbm.at[p1_v.at[s]], r1_v.at[s], sems1.at[s])
        return c0, c1

    pend = [start(0, 0), None]
    outp = [None, None]
    for k in range(nchunk):
        s = k % 2
        if k + 1 < nchunk:
            pend[(k + 1) % 2] = start(k + 1, (k + 1) % 2)
        c0, c1 = pend[s]
        c0.wait()
        c1.wait()
        if outp[s] is not None:
            outp[s].wait()
            outp[s] = None

        def body(j, _):
            for r in range(CH):
                sl = pl.ds(j * 16, 16)
                o_v[s, r, sl] = (r0_v[s, r, sl] * g0_v[s, r, :]
                                 + r1_v[s, r, sl] * g1_v[s, r, :])
            return 0

        lax.fori_loop(0, nvec, body, 0)
        base = wid * per_w + k * CH
        outp[s] = pltpu.async_copy(o_v.at[s], out_hbm.at[pl.ds(base, CH)],
                                   semo.at[s])
    for w in outp:
        if w is not None:
            w.wait()


def _combine(ys, pos0, pos1, g0r, g1r, Tt, C):
    per_w = Tt // (NC * NS)
    mesh = plsc.VectorSubcoreMesh(core_axis_name="c", subcore_axis_name="s",
                                  num_cores=NC, num_subcores=NS)
    f = functools.partial(
        pl.kernel,
        out_type=jax.ShapeDtypeStruct((Tt, C), jnp.float32),
        mesh=mesh,
        scratch_types=[
            pltpu.VMEM((2, CH), jnp.int32),
            pltpu.VMEM((2, CH), jnp.int32),
            pltpu.VMEM((2, CH, 16), jnp.float32),
            pltpu.VMEM((2, CH, 16), jnp.float32),
            pltpu.VMEM((2, CH, C), jnp.float32),
            pltpu.VMEM((2, CH, C), jnp.float32),
            pltpu.VMEM((2, CH, C), jnp.float32),
            pltpu.SemaphoreType.DMA((2,)),
            pltpu.SemaphoreType.DMA((2,)),
            pltpu.SemaphoreType.DMA((2,)),
        ],
    )(functools.partial(_combine_body, per_w, C))
    return f(ys, pos0, pos1, g0r, g1r)


def kernel(x, Wr, Wg_r, Wu_r, Wd_r, Wg_s, Wu_s, Wd_s):
    B, T, C = x.shape
    E = Wr.shape[0]
    Tt = B * T
    A = TOPK * Tt
    ntiles = -(-(A + E * (TILE - 1)) // TILE)
    M = ntiles * TILE

    xf = x.reshape(Tt, C)
    g0r, g1r, p0, p1, eot, act, zl, al = _router(xf, Wr, ntiles)

    xs = _dispatch(xf, p0.reshape(Tt), p1.reshape(Tt), M)
    ys = _mlp(xs, eot.reshape(ntiles), act.reshape(ntiles),
              Wg_s, Wu_s, Wd_s, Wg_r, Wu_r, Wd_r)
    out = _combine(ys, p0.reshape(Tt), p1.reshape(Tt), g0r, g1r, Tt, C)

    return out.reshape(B, T, C), zl.reshape(1), al.reshape(1)


# speedup vs baseline: 1.1394x; 1.1394x over previous
import functools

import jax
import jax.numpy as jnp
from jax import lax
from jax.experimental import pallas as pl
from jax.experimental.pallas import tpu as pltpu
from jax.experimental.pallas import tpu_sc as plsc

TOPK = 2
TILE = 256
CBLK = 128
NC, NS = 2, 16


def _router_body(nb, ntiles, E, S, xf_ref, wr_ref,
                 g0_ref, g1_ref, p0_ref, p1_ref, eot_ref, act_ref,
                 zl_ref, al_ref, rank_ref):
    Tt = xf_ref.shape[0]
    xf = xf_ref[...]
    wr = wr_ref[...]
    logits = lax.dot_general(xf, wr, (((1,), (1,)), ((), ())),
                             preferred_element_type=jnp.float32)
    zl_ref[...] = E * jnp.mean(jnp.square(logits), keepdims=True)

    m = jnp.max(logits, axis=1, keepdims=True)
    ex = jnp.exp(logits - m)
    probs = ex / jnp.sum(ex, axis=1, keepdims=True)
    pmean = jnp.mean(probs, axis=0, keepdims=True)
    al_ref[...] = E * (E * jnp.mean(pmean * jnp.log(pmean + 1e-9),
                                    keepdims=True))

    iota_e = lax.broadcasted_iota(jnp.int32, (Tt, E), 1)
    m0 = jnp.max(probs, axis=1, keepdims=True)
    idx0 = jnp.min(jnp.where(probs == m0, iota_e, E), axis=1, keepdims=True)
    sel0 = iota_e == idx0
    probs1 = jnp.where(sel0, -1.0, probs)
    m1 = jnp.max(probs1, axis=1, keepdims=True)
    idx1 = jnp.min(jnp.where(probs1 == m1, iota_e, E), axis=1, keepdims=True)
    sel1 = iota_e == idx1
    ssum = m0 + m1
    ones128 = jnp.ones((1, 128), jnp.float32)
    g0_ref[...] = (m0 / ssum) * ones128
    g1_ref[...] = (m1 / ssum) * ones128

    c = sel0.astype(jnp.float32) + sel1.astype(jnp.float32)
    tri = (lax.broadcasted_iota(jnp.int32, (CBLK, CBLK), 0) >
           lax.broadcasted_iota(jnp.int32, (CBLK, CBLK), 1)).astype(jnp.float32)
    off = jnp.zeros((1, E), jnp.float32)
    for b in range(nb):
        cb = c[b * CBLK:(b + 1) * CBLK, :]
        rank_ref[b * CBLK:(b + 1) * CBLK, :] = lax.dot_general(
            tri, cb, (((1,), (0,)), ((), ())),
            preferred_element_type=jnp.float32) + off
        off = off + jnp.sum(cb, axis=0, keepdims=True)
    counts = off

    padded = jnp.ceil(counts / TILE) * TILE
    triE = (lax.broadcasted_iota(jnp.int32, (E, E), 0) <
            lax.broadcasted_iota(jnp.int32, (E, E), 1)).astype(jnp.float32)
    starts = lax.dot_general(padded, triE, (((1,), (0,)), ((), ())),
                             preferred_element_type=jnp.float32)
    ends = starts + padded

    rank = rank_ref[...]
    pos0 = jnp.sum((starts + rank) * sel0.astype(jnp.float32), axis=1,
                   keepdims=True)
    pos1 = jnp.sum((starts + rank) * sel1.astype(jnp.float32), axis=1,
                   keepdims=True)
    p0_ref[...] = pos0.astype(jnp.int32)
    p1_ref[...] = pos1.astype(jnp.int32)

    tile_start = (lax.broadcasted_iota(jnp.int32, (ntiles, 1), 0)
                  * TILE).astype(jnp.float32)
    eot = jnp.sum((tile_start >= ends).astype(jnp.int32), axis=1,
                  keepdims=True)
    eot_ref[...] = jnp.minimum(eot, E - 1)
    total = jnp.sum(padded, axis=1, keepdims=True)
    act_ref[...] = (tile_start < total).astype(jnp.int32)


def _router(xf, Wr, ntiles):
    Tt, _ = xf.shape
    E = Wr.shape[0]
    S = 1
    nb = Tt // CBLK
    f32, i32 = jnp.float32, jnp.int32
    outs = pl.pallas_call(
        functools.partial(_router_body, nb, ntiles, E, S),
        out_shape=[
            jax.ShapeDtypeStruct((Tt, 128), f32),
            jax.ShapeDtypeStruct((Tt, 128), f32),
            jax.ShapeDtypeStruct((Tt, 1), i32),
            jax.ShapeDtypeStruct((Tt, 1), i32),
            jax.ShapeDtypeStruct((ntiles, 1), i32),
            jax.ShapeDtypeStruct((ntiles, 1), i32),
            jax.ShapeDtypeStruct((1, 1), f32),
            jax.ShapeDtypeStruct((1, 1), f32),
        ],
        scratch_shapes=[pltpu.VMEM((Tt, E), f32)],
    )(xf, Wr)
    return outs


def _dispatch_body(per_w, xf_hbm, p0_hbm, p1_hbm, g0_hbm, g1_hbm,
                   xs_hbm, sg_hbm,
                   p0_v, p1_v, rows_v, g0_v, g1_v, sem, sem2, sem3, sem4):
    wid = lax.axis_index("s") * NC + lax.axis_index("c")
    base = wid * per_w
    l0 = pltpu.async_copy(p0_hbm.at[pl.ds(base, per_w)], p0_v, sem)
    l1 = pltpu.async_copy(p1_hbm.at[pl.ds(base, per_w)], p1_v, sem2)
    l2 = pltpu.async_copy(xf_hbm.at[pl.ds(base, per_w)], rows_v, sem3)
    l3 = pltpu.async_copy(g0_hbm.at[pl.ds(base, per_w)], g0_v, sem4)
    l4 = pltpu.async_copy(g1_hbm.at[pl.ds(base, per_w)], g1_v, sem)
    l0.wait()
    l1.wait()
    l2.wait()
    l3.wait()
    l4.wait()
    c0 = pltpu.async_copy(rows_v, xs_hbm.at[p0_v], sem)
    c1 = pltpu.async_copy(rows_v, xs_hbm.at[p1_v], sem2)
    c2 = pltpu.async_copy(g0_v, sg_hbm.at[p0_v], sem3)
    c3 = pltpu.async_copy(g1_v, sg_hbm.at[p1_v], sem4)
    c0.wait()
    c1.wait()
    c2.wait()
    c3.wait()


def _dispatch(xf, pos0, pos1, g0r, g1r, M):
    Tt, C = xf.shape
    per_w = Tt // (NC * NS)
    mesh = plsc.VectorSubcoreMesh(core_axis_name="c", subcore_axis_name="s",
                                  num_cores=NC, num_subcores=NS)
    f = functools.partial(
        pl.kernel,
        out_type=[jax.ShapeDtypeStruct((M, C), jnp.float32),
                  jax.ShapeDtypeStruct((M, 128), jnp.float32)],
        mesh=mesh,
        scratch_types=[
            pltpu.VMEM((per_w,), jnp.int32),
            pltpu.VMEM((per_w,), jnp.int32),
            pltpu.VMEM((per_w, C), jnp.float32),
            pltpu.VMEM((per_w, 128), jnp.float32),
            pltpu.VMEM((per_w, 128), jnp.float32),
            pltpu.SemaphoreType.DMA,
            pltpu.SemaphoreType.DMA,
            pltpu.SemaphoreType.DMA,
            pltpu.SemaphoreType.DMA,
        ],
    )(functools.partial(_dispatch_body, per_w))
    return f(xf, pos0, pos1, g0r, g1r)


def _mlp_body(S, eot_ref, act_ref, xs_ref, wgs_ref, wus_ref, wds_ref,
              wgr_ref, wur_ref, wdr_ref, sg_ref, ys_ref):
    i = pl.program_id(0)
    active = act_ref[i] > 0
    is_s = eot_ref[i] < S
    xs = xs_ref[...].astype(jnp.bfloat16)

    def run(wg_ref, wu_ref, wd_ref):
        wg = wg_ref[0].astype(jnp.bfloat16)
        wu = wu_ref[0].astype(jnp.bfloat16)
        wd = wd_ref[0].astype(jnp.bfloat16)
        a = lax.dot_general(xs, wg, (((1,), (1,)), ((), ())),
                            preferred_element_type=jnp.float32)
        b = lax.dot_general(xs, wu, (((1,), (1,)), ((), ())),
                            preferred_element_type=jnp.float32)
        h = ((a * jax.nn.sigmoid(a)) * b).astype(jnp.bfloat16)
        y = lax.dot_general(h, wd, (((1,), (1,)), ((), ())),
                            preferred_element_type=jnp.float32)
        ys_ref[...] = y * sg_ref[...][:, 0:1]

    @pl.when(jnp.logical_and(active, is_s))
    def _():
        run(wgs_ref, wus_ref, wds_ref)

    @pl.when(jnp.logical_and(active, jnp.logical_not(is_s)))
    def _():
        run(wgr_ref, wur_ref, wdr_ref)


def _mlp(xs, sg, eot, act, Wg_s, Wu_s, Wd_s, Wg_r, Wu_r, Wd_r):
    M, C = xs.shape
    S, DFF, _ = Wg_s.shape
    R = Wg_r.shape[0]
    ntiles = M // TILE
    grid_spec = pltpu.PrefetchScalarGridSpec(
        num_scalar_prefetch=2,
        grid=(ntiles,),
        in_specs=[
            pl.BlockSpec((TILE, C), lambda i, eot, act: (i, 0)),
            pl.BlockSpec((1, DFF, C),
                         lambda i, eot, act: (jnp.minimum(eot[i], S - 1), 0, 0)),
            pl.BlockSpec((1, DFF, C),
                         lambda i, eot, act: (jnp.minimum(eot[i], S - 1), 0, 0)),
            pl.BlockSpec((1, C, DFF),
                         lambda i, eot, act: (jnp.minimum(eot[i], S - 1), 0, 0)),
            pl.BlockSpec((1, DFF, C),
                         lambda i, eot, act: (jnp.clip(eot[i] - S, 0, R - 1),
                                              0, 0)),
            pl.BlockSpec((1, DFF, C),
                         lambda i, eot, act: (jnp.clip(eot[i] - S, 0, R - 1),
                                              0, 0)),
            pl.BlockSpec((1, C, DFF),
                         lambda i, eot, act: (jnp.clip(eot[i] - S, 0, R - 1),
                                              0, 0)),
            pl.BlockSpec((TILE, 128), lambda i, eot, act: (i, 0)),
        ],
        out_specs=pl.BlockSpec((TILE, C), lambda i, eot, act: (i, 0)),
    )
    return pl.pallas_call(
        functools.partial(_mlp_body, S),
        grid_spec=grid_spec,
        out_shape=jax.ShapeDtypeStruct((M, C), jnp.float32),
        compiler_params=pltpu.CompilerParams(vmem_limit_bytes=63 << 20),
    )(eot, act, xs, Wg_s, Wu_s, Wd_s, Wg_r, Wu_r, Wd_r, sg)


CH = 16


def _combine_body(per_w, C, ys_hbm, p0_hbm, p1_hbm, out_hbm,
                  p0_v, p1_v, r0_v, r1_v, o_v, sems0, sems1, semo):
    wid = lax.axis_index("s") * NC + lax.axis_index("c")
    nchunk = per_w // CH
    nvec = C // 16

    def start(k, s):
        base = wid * per_w + k * CH
        pltpu.sync_copy(p0_hbm.at[pl.ds(base, CH)], p0_v.at[s])
        pltpu.sync_copy(p1_hbm.at[pl.ds(base, CH)], p1_v.at[s])
        c0 = pltpu.async_copy(ys_hbm.at[p0_v.at[s]], r0_v.at[s], sems0.at[s])
        c1 = pltpu.async_copy(ys_hbm.at[p1_v.at[s]], r1_v.at[s], sems1.at[s])
        return c0, c1

    pend = [start(0, 0), None]
    outp = [None, None]
    for k in range(nchunk):
        s = k % 2
        if k + 1 < nchunk:
            pend[(k + 1) % 2] = start(k + 1, (k + 1) % 2)
        c0, c1 = pend[s]
        c0.wait()
        c1.wait()
        if outp[s] is not None:
            outp[s].wait()
            outp[s] = None

        def body(j, _):
            for r in range(CH):
                sl = pl.ds(j * 16, 16)
                o_v[s, r, sl] = r0_v[s, r, sl] + r1_v[s, r, sl]
            return 0

        lax.fori_loop(0, nvec, body, 0)
        base = wid * per_w + k * CH
        outp[s] = pltpu.async_copy(o_v.at[s], out_hbm.at[pl.ds(base, CH)],
                                   semo.at[s])
    for w in outp:
        if w is not None:
            w.wait()


def _combine(ys, pos0, pos1, Tt, C):
    per_w = Tt // (NC * NS)
    mesh = plsc.VectorSubcoreMesh(core_axis_name="c", subcore_axis_name="s",
                                  num_cores=NC, num_subcores=NS)
    f = functools.partial(
        pl.kernel,
        out_type=jax.ShapeDtypeStruct((Tt, C), jnp.float32),
        mesh=mesh,
        scratch_types=[
            pltpu.VMEM((2, CH), jnp.int32),
            pltpu.VMEM((2, CH), jnp.int32),
            pltpu.VMEM((2, CH, C), jnp.float32),
            pltpu.VMEM((2, CH, C), jnp.float32),
            pltpu.VMEM((2, CH, C), jnp.float32),
            pltpu.SemaphoreType.DMA((2,)),
            pltpu.SemaphoreType.DMA((2,)),
            pltpu.SemaphoreType.DMA((2,)),
        ],
    )(functools.partial(_combine_body, per_w, C))
    return f(ys, pos0, pos1)


def kernel(x, Wr, Wg_r, Wu_r, Wd_r, Wg_s, Wu_s, Wd_s):
    B, T, C = x.shape
    E = Wr.shape[0]
    Tt = B * T
    A = TOPK * Tt
    ntiles = -(-(A + E * (TILE - 1)) // TILE)
    M = ntiles * TILE

    xf = x.reshape(Tt, C)
    g0r, g1r, p0, p1, eot, act, zl, al = _router(xf, Wr, ntiles)

    xs, sg = _dispatch(xf, p0.reshape(Tt), p1.reshape(Tt), g0r, g1r, M)
    ys = _mlp(xs, sg, eot.reshape(ntiles), act.reshape(ntiles),
              Wg_s, Wu_s, Wd_s, Wg_r, Wu_r, Wd_r)
    out = _combine(ys, p0.reshape(Tt), p1.reshape(Tt), Tt, C)

    return out.reshape(B, T, C), zl.reshape(1), al.reshape(1)


# speedup vs baseline: 1.1785x; 1.0343x over previous
import functools

import jax
import jax.numpy as jnp
from jax import lax
from jax.experimental import pallas as pl
from jax.experimental.pallas import tpu as pltpu
from jax.experimental.pallas import tpu_sc as plsc

TOPK = 2
TILE = 256
CBLK = 128
NC, NS = 2, 16


def _router_body(nb, ntiles, E, S, xf_ref, wr_ref,
                 g0_ref, g1_ref, p0_ref, p1_ref, eot_ref, act_ref,
                 zl_ref, al_ref, rank_ref):
    Tt = xf_ref.shape[0]
    xf = xf_ref[...]
    wr = wr_ref[...]
    logits = lax.dot_general(xf, wr, (((1,), (1,)), ((), ())),
                             preferred_element_type=jnp.float32)
    zl_ref[...] = E * jnp.mean(jnp.square(logits), keepdims=True)

    m = jnp.max(logits, axis=1, keepdims=True)
    ex = jnp.exp(logits - m)
    probs = ex / jnp.sum(ex, axis=1, keepdims=True)
    pmean = jnp.mean(probs, axis=0, keepdims=True)
    al_ref[...] = E * (E * jnp.mean(pmean * jnp.log(pmean + 1e-9),
                                    keepdims=True))

    iota_e = lax.broadcasted_iota(jnp.int32, (Tt, E), 1)
    m0 = jnp.max(probs, axis=1, keepdims=True)
    idx0 = jnp.min(jnp.where(probs == m0, iota_e, E), axis=1, keepdims=True)
    sel0 = iota_e == idx0
    probs1 = jnp.where(sel0, -1.0, probs)
    m1 = jnp.max(probs1, axis=1, keepdims=True)
    idx1 = jnp.min(jnp.where(probs1 == m1, iota_e, E), axis=1, keepdims=True)
    sel1 = iota_e == idx1
    ssum = m0 + m1
    ones128 = jnp.ones((1, 128), jnp.float32)
    g0_ref[...] = (m0 / ssum) * ones128
    g1_ref[...] = (m1 / ssum) * ones128

    c = sel0.astype(jnp.float32) + sel1.astype(jnp.float32)
    tri = (lax.broadcasted_iota(jnp.int32, (CBLK, CBLK), 0) >
           lax.broadcasted_iota(jnp.int32, (CBLK, CBLK), 1)).astype(jnp.float32)
    off = jnp.zeros((1, E), jnp.float32)
    for b in range(nb):
        cb = c[b * CBLK:(b + 1) * CBLK, :]
        rank_ref[b * CBLK:(b + 1) * CBLK, :] = lax.dot_general(
            tri, cb, (((1,), (0,)), ((), ())),
            preferred_element_type=jnp.float32) + off
        off = off + jnp.sum(cb, axis=0, keepdims=True)
    counts = off

    padded = jnp.ceil(counts / TILE) * TILE
    triE = (lax.broadcasted_iota(jnp.int32, (E, E), 0) <
            lax.broadcasted_iota(jnp.int32, (E, E), 1)).astype(jnp.float32)
    starts = lax.dot_general(padded, triE, (((1,), (0,)), ((), ())),
                             preferred_element_type=jnp.float32)
    ends = starts + padded

    rank = rank_ref[...]
    pos0 = jnp.sum((starts + rank) * sel0.astype(jnp.float32), axis=1,
                   keepdims=True)
    pos1 = jnp.sum((starts + rank) * sel1.astype(jnp.float32), axis=1,
                   keepdims=True)
    p0_ref[...] = jnp.swapaxes(pos0.astype(jnp.int32), 0, 1)
    p1_ref[...] = jnp.swapaxes(pos1.astype(jnp.int32), 0, 1)

    tile_start = (lax.broadcasted_iota(jnp.int32, (ntiles, 1), 0)
                  * TILE).astype(jnp.float32)
    eot = jnp.sum((tile_start >= ends).astype(jnp.int32), axis=1,
                  keepdims=True)
    eot_ref[...] = jnp.minimum(eot, E - 1)
    total = jnp.sum(padded, axis=1, keepdims=True)
    act_ref[...] = (tile_start < total).astype(jnp.int32)


def _router(xf, Wr, ntiles):
    Tt, _ = xf.shape
    E = Wr.shape[0]
    S = 1
    nb = Tt // CBLK
    f32, i32 = jnp.float32, jnp.int32
    outs = pl.pallas_call(
        functools.partial(_router_body, nb, ntiles, E, S),
        out_shape=[
            jax.ShapeDtypeStruct((Tt, 128), f32),
            jax.ShapeDtypeStruct((Tt, 128), f32),
            jax.ShapeDtypeStruct((1, Tt), i32),
            jax.ShapeDtypeStruct((1, Tt), i32),
            jax.ShapeDtypeStruct((ntiles, 1), i32),
            jax.ShapeDtypeStruct((ntiles, 1), i32),
            jax.ShapeDtypeStruct((1, 1), f32),
            jax.ShapeDtypeStruct((1, 1), f32),
        ],
        scratch_shapes=[pltpu.VMEM((Tt, E), f32)],
    )(xf, Wr)
    return outs


def _dispatch_body(per_w, xf_hbm, p0_hbm, p1_hbm, g0_hbm, g1_hbm,
                   xs_hbm, sg_hbm,
                   p0_v, p1_v, rows_v, g0_v, g1_v, sem, sem2, sem3, sem4):
    wid = lax.axis_index("s") * NC + lax.axis_index("c")
    base = wid * per_w
    l0 = pltpu.async_copy(p0_hbm.at[pl.ds(base, per_w)], p0_v, sem)
    l1 = pltpu.async_copy(p1_hbm.at[pl.ds(base, per_w)], p1_v, sem2)
    l2 = pltpu.async_copy(xf_hbm.at[pl.ds(base, per_w)], rows_v, sem3)
    l3 = pltpu.async_copy(g0_hbm.at[pl.ds(base, per_w)], g0_v, sem4)
    l4 = pltpu.async_copy(g1_hbm.at[pl.ds(base, per_w)], g1_v, sem)
    l0.wait()
    l1.wait()
    l2.wait()
    l3.wait()
    l4.wait()
    c0 = pltpu.async_copy(rows_v, xs_hbm.at[p0_v], sem)
    c1 = pltpu.async_copy(rows_v, xs_hbm.at[p1_v], sem2)
    c2 = pltpu.async_copy(g0_v, sg_hbm.at[p0_v], sem3)
    c3 = pltpu.async_copy(g1_v, sg_hbm.at[p1_v], sem4)
    c0.wait()
    c1.wait()
    c2.wait()
    c3.wait()


def _dispatch(xf, pos0, pos1, g0r, g1r, M):
    Tt, C = xf.shape
    per_w = Tt // (NC * NS)
    mesh = plsc.VectorSubcoreMesh(core_axis_name="c", subcore_axis_name="s",
                                  num_cores=NC, num_subcores=NS)
    f = functools.partial(
        pl.kernel,
        out_type=[jax.ShapeDtypeStruct((M, C), jnp.float32),
                  jax.ShapeDtypeStruct((M, 128), jnp.float32)],
        mesh=mesh,
        scratch_types=[
            pltpu.VMEM((per_w,), jnp.int32),
            pltpu.VMEM((per_w,), jnp.int32),
            pltpu.VMEM((per_w, C), jnp.float32),
            pltpu.VMEM((per_w, 128), jnp.float32),
            pltpu.VMEM((per_w, 128), jnp.float32),
            pltpu.SemaphoreType.DMA,
            pltpu.SemaphoreType.DMA,
            pltpu.SemaphoreType.DMA,
            pltpu.SemaphoreType.DMA,
        ],
    )(functools.partial(_dispatch_body, per_w))
    return f(xf, pos0, pos1, g0r, g1r)


def _mlp_body(S, eot_ref, act_ref, xs_ref, wgs_ref, wus_ref, wds_ref,
              wgr_ref, wur_ref, wdr_ref, sg_ref, ys_ref):
    i = pl.program_id(0)
    active = act_ref[i] > 0
    is_s = eot_ref[i] < S
    xs = xs_ref[...].astype(jnp.bfloat16)

    def run(wg_ref, wu_ref, wd_ref):
        wg = wg_ref[0].astype(jnp.bfloat16)
        wu = wu_ref[0].astype(jnp.bfloat16)
        wd = wd_ref[0].astype(jnp.bfloat16)
        a = lax.dot_general(xs, wg, (((1,), (1,)), ((), ())),
                            preferred_element_type=jnp.float32)
        b = lax.dot_general(xs, wu, (((1,), (1,)), ((), ())),
                            preferred_element_type=jnp.float32)
        h = ((a * jax.nn.sigmoid(a)) * b).astype(jnp.bfloat16)
        y = lax.dot_general(h, wd, (((1,), (1,)), ((), ())),
                            preferred_element_type=jnp.float32)
        ys_ref[...] = y * sg_ref[...][:, 0:1]

    @pl.when(jnp.logical_and(active, is_s))
    def _():
        run(wgs_ref, wus_ref, wds_ref)

    @pl.when(jnp.logical_and(active, jnp.logical_not(is_s)))
    def _():
        run(wgr_ref, wur_ref, wdr_ref)


def _mlp(xs, sg, eot, act, Wg_s, Wu_s, Wd_s, Wg_r, Wu_r, Wd_r):
    M, C = xs.shape
    S, DFF, _ = Wg_s.shape
    R = Wg_r.shape[0]
    ntiles = M // TILE
    grid_spec = pltpu.PrefetchScalarGridSpec(
        num_scalar_prefetch=2,
        grid=(ntiles,),
        in_specs=[
            pl.BlockSpec((TILE, C), lambda i, eot, act: (i, 0)),
            pl.BlockSpec((1, DFF, C),
                         lambda i, eot, act: (jnp.minimum(eot[i], S - 1), 0, 0)),
            pl.BlockSpec((1, DFF, C),
                         lambda i, eot, act: (jnp.minimum(eot[i], S - 1), 0, 0)),
            pl.BlockSpec((1, C, DFF),
                         lambda i, eot, act: (jnp.minimum(eot[i], S - 1), 0, 0)),
            pl.BlockSpec((1, DFF, C),
                         lambda i, eot, act: (jnp.clip(eot[i] - S, 0, R - 1),
                                              0, 0)),
            pl.BlockSpec((1, DFF, C),
                         lambda i, eot, act: (jnp.clip(eot[i] - S, 0, R - 1),
                                              0, 0)),
            pl.BlockSpec((1, C, DFF),
                         lambda i, eot, act: (jnp.clip(eot[i] - S, 0, R - 1),
                                              0, 0)),
            pl.BlockSpec((TILE, 128), lambda i, eot, act: (i, 0)),
        ],
        out_specs=pl.BlockSpec((TILE, C), lambda i, eot, act: (i, 0)),
    )
    return pl.pallas_call(
        functools.partial(_mlp_body, S),
        grid_spec=grid_spec,
        out_shape=jax.ShapeDtypeStruct((M, C), jnp.float32),
        compiler_params=pltpu.CompilerParams(vmem_limit_bytes=63 << 20),
    )(eot, act, xs, Wg_s, Wu_s, Wd_s, Wg_r, Wu_r, Wd_r, sg)


CH = 16


def _combine_body(per_w, C, ys_hbm, p0_hbm, p1_hbm, out_hbm,
                  p0_v, p1_v, r0_v, r1_v, o_v, sems0, sems1, semo):
    wid = lax.axis_index("s") * NC + lax.axis_index("c")
    nchunk = per_w // CH
    nvec = C // 16

    def start(k, s):
        base = wid * per_w + k * CH
        pltpu.sync_copy(p0_hbm.at[pl.ds(base, CH)], p0_v.at[s])
        pltpu.sync_copy(p1_hbm.at[pl.ds(base, CH)], p1_v.at[s])
        c0 = pltpu.async_copy(ys_hbm.at[p0_v.at[s]], r0_v.at[s], sems0.at[s])
        c1 = pltpu.async_copy(ys_hbm.at[p1_v.at[s]], r1_v.at[s], sems1.at[s])
        return c0, c1

    pend = [start(0, 0), None]
    outp = [None, None]
    for k in range(nchunk):
        s = k % 2
        if k + 1 < nchunk:
            pend[(k + 1) % 2] = start(k + 1, (k + 1) % 2)
        c0, c1 = pend[s]
        c0.wait()
        c1.wait()
        if outp[s] is not None:
            outp[s].wait()
            outp[s] = None

        def body(j, _):
            for r in range(CH):
                sl = pl.ds(j * 16, 16)
                o_v[s, r, sl] = r0_v[s, r, sl] + r1_v[s, r, sl]
            return 0

        lax.fori_loop(0, nvec, body, 0)
        base = wid * per_w + k * CH
        outp[s] = pltpu.async_copy(o_v.at[s], out_hbm.at[pl.ds(base, CH)],
                                   semo.at[s])
    for w in outp:
        if w is not None:
            w.wait()


def _combine(ys, pos0, pos1, Tt, C):
    per_w = Tt // (NC * NS)
    mesh = plsc.VectorSubcoreMesh(core_axis_name="c", subcore_axis_name="s",
                                  num_cores=NC, num_subcores=NS)
    f = functools.partial(
        pl.kernel,
        out_type=jax.ShapeDtypeStruct((Tt, C), jnp.float32),
        mesh=mesh,
        scratch_types=[
            pltpu.VMEM((2, CH), jnp.int32),
            pltpu.VMEM((2, CH), jnp.int32),
            pltpu.VMEM((2, CH, C), jnp.float32),
            pltpu.VMEM((2, CH, C), jnp.float32),
            pltpu.VMEM((2, CH, C), jnp.float32),
            pltpu.SemaphoreType.DMA((2,)),
            pltpu.SemaphoreType.DMA((2,)),
            pltpu.SemaphoreType.DMA((2,)),
        ],
    )(functools.partial(_combine_body, per_w, C))
    return f(ys, pos0, pos1)


def kernel(x, Wr, Wg_r, Wu_r, Wd_r, Wg_s, Wu_s, Wd_s):
    B, T, C = x.shape
    E = Wr.shape[0]
    Tt = B * T
    A = TOPK * Tt
    ntiles = -(-(A + E * (TILE - 1)) // TILE)
    M = ntiles * TILE

    xf = x.reshape(Tt, C)
    g0r, g1r, p0, p1, eot, act, zl, al = _router(xf, Wr, ntiles)

    xs, sg = _dispatch(xf, p0.reshape(Tt), p1.reshape(Tt), g0r, g1r, M)
    ys = _mlp(xs, sg, eot.reshape(ntiles), act.reshape(ntiles),
              Wg_s, Wu_s, Wd_s, Wg_r, Wu_r, Wd_r)
    out = _combine(ys, p0.reshape(Tt), p1.reshape(Tt), Tt, C)

    return out.reshape(B, T, C), zl.reshape(1), al.reshape(1)
